# baseline (speedup 1.0000x reference)
import functools
import jax
import jax.numpy as jnp
from jax import lax
from jax.experimental import pallas as pl
from jax.experimental.pallas import tpu as pltpu
from jax.experimental.pallas import tpu_sc as plsc

QP = 64
C = 64
NPARAM = 11
ROW = 768


def _transform_body(t_ref, o_ref):
    sp = [jax.nn.softplus(t_ref[:, i, :]) for i in (0, 3, 6, 9)]
    ta = [jnp.tanh(t_ref[:, i, :]) for i in (2, 5, 8)]
    b = [t_ref[:, i, :] for i in (1, 4, 7, 10)]

    def put(i, v):
        o_ref[:, i * C:(i + 1) * C] = v

    put(0, sp[0])
    put(1, b[0])
    for layer in range(3):
        scale = sp[layer + 1] if layer < 2 else 0.5 * sp[3]
        put(3 * layer + 2, scale)
        put(3 * layer + 3, ta[layer] * scale)
        put(3 * layer + 4, b[layer + 1] * (1.0 if layer < 2 else 0.5))
    o_ref[:, NPARAM * C:] = jnp.zeros((QP, ROW - NPARAM * C), jnp.float32)


def _sc_gather(ttable, index, num_cores):
    B = index.shape[0]
    mesh = plsc.VectorSubcoreMesh(core_axis_name="c", subcore_axis_name="s")

    @functools.partial(
        pl.kernel,
        mesh=mesh,
        out_type=jax.ShapeDtypeStruct((B, ROW), jnp.float32),
        scratch_types=[
            pltpu.VMEM((B,), jnp.int32),
            pltpu.VMEM((B, ROW), jnp.float32),
            pltpu.SemaphoreType.DMA,
        ],
    )
    def gather_kernel(table_hbm, idx_hbm, out_hbm, idx_v, rows_v, sem):
        wid = lax.axis_index("s") * num_cores + lax.axis_index("c")

        @pl.when(wid == 0)
        def _():
            pltpu.sync_copy(idx_hbm, idx_v)
            pltpu.async_copy(table_hbm.at[idx_v], rows_v, sem).wait()
            pltpu.sync_copy(rows_v, out_hbm)

    return gather_kernel(ttable, index)


def _main_body(p_ref, x_ref, o_ref):
    b = pl.program_id(0)

    def chan(c, _):
        y = x_ref[0, c]

        def s(i):
            return p_ref[b, i * C + c]

        y = y * s(0) + s(1)
        for layer in range(3):
            y = y * s(3 * layer + 2) + jnp.tanh(y) * s(3 * layer + 3) + s(3 * layer + 4)
        o_ref[0, c] = 0.5 * jnp.tanh(y) + 0.5
        return 0

    jax.lax.fori_loop(0, x_ref.shape[1], chan, 0, unroll=32)


@jax.jit
def kernel(x, index, h1, b1, a1, h2, b2, a2, h3, b3, a3, h4, b4):
    B, Cx, H, W = x.shape
    table = jnp.stack(
        [t.reshape(QP, C) for t in (h1, b1, a1, h2, b2, a2, h3, b3, a3, h4, b4)],
        axis=1,
    )

    ttable = pl.pallas_call(
        _transform_body,
        out_shape=jax.ShapeDtypeStruct((QP, ROW), x.dtype),
    )(table)

    info = plsc.get_sparse_core_info()
    params = _sc_gather(ttable, index, info.num_cores)

    return pl.pallas_call(
        _main_body,
        grid=(B,),
        in_specs=[
            pl.BlockSpec(memory_space=pltpu.SMEM),
            pl.BlockSpec((1, Cx, H, W), lambda b: (b, 0, 0, 0)),
        ],
        out_specs=pl.BlockSpec((1, Cx, H, W), lambda b: (b, 0, 0, 0)),
        out_shape=jax.ShapeDtypeStruct(x.shape, x.dtype),
    )(params, x)
